# baseline (speedup 1.0000x reference)
import jax
import jax.numpy as jnp
from jax.experimental import pallas as pl
from jax.experimental.pallas import tpu as pltpu

_BLOCK_ROWS = 5000


def _dual_read_probe(a_ref, b_ref, o_ref):
    i = pl.program_id(0)

    @pl.when(i == 0)
    def _init():
        o_ref[...] = jnp.zeros_like(o_ref)

    o_ref[...] += jnp.sum(a_ref[...], axis=0, keepdims=True)
    o_ref[...] += jnp.sum(b_ref[...], axis=0, keepdims=True)


def kernel(x, edge_index):
    del edge_index
    n_rows, d = x.shape
    grid = (n_rows // _BLOCK_ROWS // 2,)
    return pl.pallas_call(
        _dual_read_probe,
        grid=grid,
        in_specs=[
            pl.BlockSpec((_BLOCK_ROWS, d), lambda i: (2 * i, 0)),
            pl.BlockSpec((_BLOCK_ROWS, d), lambda i: (2 * i + 1, 0)),
        ],
        out_specs=pl.BlockSpec((1, d), lambda i: (0, 0)),
        out_shape=jax.ShapeDtypeStruct((1, d), x.dtype),
    )(x, x)
